# baseline (speedup 1.0000x reference)
import functools

import jax
import jax.numpy as jnp
from jax.experimental import pallas as pl
from jax.experimental.pallas import tpu as pltpu

_MAXK = 128
_BISECT_ITERS = 46


def _fps_body(px_ref, py_ref, qx_ref, qy_ref, *, M):
    px = px_ref[...]
    py = py_ref[...]
    B, N = px.shape
    iota_n = jax.lax.broadcasted_iota(jnp.int32, (1, N), 1)
    iota_m = jax.lax.broadcasted_iota(jnp.int32, (1, M), 1)
    lastx = px[:, 0:1]
    lasty = py[:, 0:1]
    sel0 = iota_m == 0
    qx = jnp.where(sel0, lastx, 0.0)
    qy = jnp.where(sel0, lasty, 0.0)
    dist0 = jnp.full((B, N), jnp.inf, dtype=jnp.float32)

    def body(i, carry):
        dist, lx, ly, qx, qy = carry
        d = (px - lx) ** 2 + (py - ly) ** 2
        dist = jnp.minimum(dist, d)
        m = jnp.max(dist, axis=1, keepdims=True)
        idx = jnp.min(jnp.where(dist == m, iota_n, N), axis=1, keepdims=True)
        selp = iota_n == idx
        nx = jnp.sum(jnp.where(selp, px, 0.0), axis=1, keepdims=True)
        ny = jnp.sum(jnp.where(selp, py, 0.0), axis=1, keepdims=True)
        selq = iota_m == i
        qx = jnp.where(selq, nx, qx)
        qy = jnp.where(selq, ny, qy)
        return dist, nx, ny, qx, qy

    _, _, _, qx, qy = jax.lax.fori_loop(1, M, body, (dist0, lastx, lasty, qx, qy))
    qx_ref[...] = qx
    qy_ref[...] = qy


def _fps(px, py, M):
    B, N = px.shape
    return pl.pallas_call(
        functools.partial(_fps_body, M=M),
        out_shape=[
            jax.ShapeDtypeStruct((B, M), jnp.float32),
            jax.ShapeDtypeStruct((B, M), jnp.float32),
        ],
    )(px, py)


def _sa_body(xT_ref, posT_ref, q_ref, qT_ref, waxT_ref, wapT_ref, baT_ref,
             wbT_ref, bbT_ref, out_ref, *, r2):
    XT = xT_ref[0]
    PT = posT_ref[0]
    Q = q_ref[0]
    QT = qT_ref[0]
    N = XT.shape[1]
    M = QT.shape[1]
    f32 = jnp.float32

    PUT = (jnp.dot(waxT_ref[...], XT, preferred_element_type=f32)
           + jnp.dot(wapT_ref[...], PT, preferred_element_type=f32)
           + baT_ref[...])
    VT = jnp.dot(wapT_ref[...], QT, preferred_element_type=f32)

    PTx = PT[0:1, :]
    PTy = PT[1:2, :]
    QTx = QT[0:1, :]
    QTy = QT[1:2, :]
    Qx = Q[:, 0:1]
    Qy = Q[:, 1:2]

    ddx = Qx - PTx
    ddy = Qy - PTy
    D = ddx * ddx + ddy * ddy
    cnt = jnp.sum((D <= r2).astype(jnp.int32), axis=1, keepdims=True)

    def bis(_, c):
        lo, hi = c
        mid = 0.5 * (lo + hi)
        cm = jnp.sum((D <= mid).astype(jnp.int32), axis=1, keepdims=True)
        ge = cm >= _MAXK
        return jnp.where(ge, lo, mid), jnp.where(ge, mid, hi)

    lo0 = jnp.zeros((M, 1), f32)
    hi0 = jnp.full((M, 1), r2, f32)
    _, hi = jax.lax.fori_loop(0, _BISECT_ITERS, bis, (lo0, hi0))
    thresh = jnp.where(cnt > _MAXK, hi, jnp.full((M, 1), r2, f32))

    WbT = wbT_ref[...].astype(jnp.bfloat16)
    bbT = bbT_ref[...]
    Co2 = WbT.shape[0]
    iota_m = jax.lax.broadcasted_iota(jnp.int32, (1, M), 1)
    iota_mc = jax.lax.broadcasted_iota(jnp.int32, (M, 1), 0)

    def qloop(q, acc):
        sel = iota_m == q
        qx = jnp.sum(jnp.where(sel, QTx, 0.0))
        qy = jnp.sum(jnp.where(sel, QTy, 0.0))
        th = jnp.sum(jnp.where(iota_mc == q, thresh, 0.0))
        ex = qx - PTx
        ey = qy - PTy
        bias = jnp.where(ex * ex + ey * ey <= th, 0.0, -1e30)
        vcol = jnp.sum(jnp.where(sel, VT, 0.0), axis=1, keepdims=True)
        tT = jnp.tanh(PUT - vcol).astype(jnp.bfloat16)
        hT = jnp.dot(WbT, tT, preferred_element_type=f32)
        r = jnp.max(hT + bias, axis=1, keepdims=True) + bbT
        return jnp.where(sel, r, acc)

    acc0 = jnp.zeros((Co2, M), f32)
    out_ref[0] = jax.lax.fori_loop(0, M, qloop, acc0)


def _sa(XT, posT, q, qT, Wa, ba, Wb, bb, r2):
    B, F, N = XT.shape
    M = qT.shape[2]
    Co2 = Wb.shape[1]
    waxT = Wa[:F].T
    wapT = Wa[F:].T
    baT = ba.reshape(-1, 1)
    wbT = Wb.T
    bbT = bb.reshape(-1, 1)
    return pl.pallas_call(
        functools.partial(_sa_body, r2=r2),
        grid=(B,),
        in_specs=[
            pl.BlockSpec((1, F, N), lambda b: (b, 0, 0)),
            pl.BlockSpec((1, 2, N), lambda b: (b, 0, 0)),
            pl.BlockSpec((1, M, 2), lambda b: (b, 0, 0)),
            pl.BlockSpec((1, 2, M), lambda b: (b, 0, 0)),
            pl.BlockSpec(waxT.shape, lambda b: (0, 0)),
            pl.BlockSpec(wapT.shape, lambda b: (0, 0)),
            pl.BlockSpec(baT.shape, lambda b: (0, 0)),
            pl.BlockSpec(wbT.shape, lambda b: (0, 0)),
            pl.BlockSpec(bbT.shape, lambda b: (0, 0)),
        ],
        out_specs=pl.BlockSpec((1, Co2, M), lambda b: (b, 0, 0)),
        out_shape=jax.ShapeDtypeStruct((B, Co2, M), jnp.float32),
        compiler_params=pltpu.CompilerParams(
            dimension_semantics=("parallel",)),
    )(XT, posT, q, qT, waxT, wapT, baT, wbT, bbT)


def _glob_body(xT_ref, qT_ref, waxT_ref, wapT_ref, baT_ref, wbT_ref, bbT_ref,
               out_ref):
    f32 = jnp.float32
    XT = xT_ref[0]
    QT = qT_ref[0]
    hT = jnp.tanh(jnp.dot(waxT_ref[...], XT, preferred_element_type=f32)
                  + jnp.dot(wapT_ref[...], QT, preferred_element_type=f32)
                  + baT_ref[...])
    oT = jnp.dot(wbT_ref[...], hT, preferred_element_type=f32) + bbT_ref[...]
    out_ref[0] = jnp.max(oT, axis=1, keepdims=True)


def _glob(XT, qT, Wa, ba, Wb, bb):
    B, C, M = XT.shape
    Co2 = Wb.shape[1]
    waxT = Wa[:C].T
    wapT = Wa[C:].T
    baT = ba.reshape(-1, 1)
    wbT = Wb.T
    bbT = bb.reshape(-1, 1)
    return pl.pallas_call(
        _glob_body,
        grid=(B,),
        in_specs=[
            pl.BlockSpec((1, C, M), lambda b: (b, 0, 0)),
            pl.BlockSpec((1, 2, M), lambda b: (b, 0, 0)),
            pl.BlockSpec(waxT.shape, lambda b: (0, 0)),
            pl.BlockSpec(wapT.shape, lambda b: (0, 0)),
            pl.BlockSpec(baT.shape, lambda b: (0, 0)),
            pl.BlockSpec(wbT.shape, lambda b: (0, 0)),
            pl.BlockSpec(bbT.shape, lambda b: (0, 0)),
        ],
        out_specs=pl.BlockSpec((1, Co2, 1), lambda b: (b, 0, 0)),
        out_shape=jax.ShapeDtypeStruct((B, Co2, 1), jnp.float32),
        compiler_params=pltpu.CompilerParams(
            dimension_semantics=("parallel",)),
    )(XT, qT, waxT, wapT, baT, wbT, bbT)[:, :, 0]


def kernel(x, pos, W1a, b1a, W1b, b1b, W2a, b2a, W2b, b2b, W3a, b3a, W3b, b3b):
    B, N, _ = x.shape
    M1 = N // 4
    M2 = M1 // 4
    r1sq = float(0.4 * 0.4)
    r2sq = float(0.8 * 0.8)

    px = pos[:, :, 0]
    py = pos[:, :, 1]
    xT = jnp.transpose(x, (0, 2, 1))
    posT = jnp.stack([px, py], axis=1)

    q1x, q1y = _fps(px, py, M1)
    q1 = jnp.stack([q1x, q1y], axis=-1)
    q1T = jnp.stack([q1x, q1y], axis=1)

    x1T = _sa(xT, posT, q1, q1T, W1a, b1a, W1b, b1b, r2=r1sq)

    q2x, q2y = _fps(q1x, q1y, M2)
    q2 = jnp.stack([q2x, q2y], axis=-1)
    q2T = jnp.stack([q2x, q2y], axis=1)

    x2T = _sa(x1T, q1T, q2, q2T, W2a, b2a, W2b, b2b, r2=r2sq)

    return _glob(x2T, q2T, W3a, b3a, W3b, b3b)


# speedup vs baseline: 1.0832x; 1.0832x over previous
import functools

import jax
import jax.numpy as jnp
from jax.experimental import pallas as pl
from jax.experimental.pallas import tpu as pltpu

_MAXK = 128
_BISECT_ITERS = 46


def _fps_body(px_ref, py_ref, qx_ref, qy_ref, *, M):
    px = px_ref[...]
    py = py_ref[...]
    B, N = px.shape
    iota_n = jax.lax.broadcasted_iota(jnp.int32, (1, N), 1)
    iota_m = jax.lax.broadcasted_iota(jnp.int32, (1, M), 1)
    lastx = px[:, 0:1]
    lasty = py[:, 0:1]
    sel0 = iota_m == 0
    qx = jnp.where(sel0, lastx, 0.0)
    qy = jnp.where(sel0, lasty, 0.0)
    dist0 = jnp.full((B, N), jnp.inf, dtype=jnp.float32)

    def body(i, carry):
        dist, lx, ly, qx, qy = carry
        d = (px - lx) ** 2 + (py - ly) ** 2
        dist = jnp.minimum(dist, d)
        m = jnp.max(dist, axis=1, keepdims=True)
        idx = jnp.min(jnp.where(dist == m, iota_n, N), axis=1, keepdims=True)
        selp = iota_n == idx
        nx = jnp.sum(jnp.where(selp, px, 0.0), axis=1, keepdims=True)
        ny = jnp.sum(jnp.where(selp, py, 0.0), axis=1, keepdims=True)
        selq = iota_m == i
        qx = jnp.where(selq, nx, qx)
        qy = jnp.where(selq, ny, qy)
        return dist, nx, ny, qx, qy

    _, _, _, qx, qy = jax.lax.fori_loop(1, M, body, (dist0, lastx, lasty, qx, qy))
    qx_ref[...] = qx
    qy_ref[...] = qy


def _fps(px, py, M):
    B, N = px.shape
    return pl.pallas_call(
        functools.partial(_fps_body, M=M),
        out_shape=[
            jax.ShapeDtypeStruct((B, M), jnp.float32),
            jax.ShapeDtypeStruct((B, M), jnp.float32),
        ],
    )(px, py)


def _sa_body(xT_ref, posT_ref, q_ref, qT_ref, waxT_ref, wapT_ref, baT_ref,
             wbT_ref, bbT_ref, out_ref, *, r2):
    XT = xT_ref[0]
    PT = posT_ref[0]
    Q = q_ref[0]
    QT = qT_ref[0]
    N = XT.shape[1]
    M = QT.shape[1]
    f32 = jnp.float32

    PUT = (jnp.dot(waxT_ref[...], XT, preferred_element_type=f32)
           + jnp.dot(wapT_ref[...], PT, preferred_element_type=f32)
           + baT_ref[...])
    VT = jnp.dot(wapT_ref[...], QT, preferred_element_type=f32)

    PTx = PT[0:1, :]
    PTy = PT[1:2, :]
    QTx = QT[0:1, :]
    QTy = QT[1:2, :]
    Qx = Q[:, 0:1]
    Qy = Q[:, 1:2]

    ddx = Qx - PTx
    ddy = Qy - PTy
    D = ddx * ddx + ddy * ddy
    cnt = jnp.sum((D <= r2).astype(jnp.int32), axis=1, keepdims=True)

    def bis(_, c):
        lo, hi = c
        mid = 0.5 * (lo + hi)
        cm = jnp.sum((D <= mid).astype(jnp.int32), axis=1, keepdims=True)
        ge = cm >= _MAXK
        return jnp.where(ge, lo, mid), jnp.where(ge, mid, hi)

    lo0 = jnp.zeros((M, 1), f32)
    hi0 = jnp.full((M, 1), r2, f32)
    _, hi = jax.lax.fori_loop(0, _BISECT_ITERS, bis, (lo0, hi0))
    thresh = jnp.where(cnt > _MAXK, hi, jnp.full((M, 1), r2, f32))

    WbT = wbT_ref[...].astype(jnp.bfloat16)
    bbT = bbT_ref[...]
    Co2 = WbT.shape[0]
    bf16 = jnp.bfloat16
    iota_m = jax.lax.broadcasted_iota(jnp.int32, (1, M), 1)
    iota_mc = jax.lax.broadcasted_iota(jnp.int32, (M, 1), 0)

    def stage_a(q):
        sel = iota_m == q
        qx = jnp.sum(jnp.where(sel, QTx, 0.0))
        qy = jnp.sum(jnp.where(sel, QTy, 0.0))
        th = jnp.sum(jnp.where(iota_mc == q, thresh, 0.0))
        ex = qx - PTx
        ey = qy - PTy
        bias = jnp.where(ex * ex + ey * ey <= th, 0.0, -1e30)
        vcol = jnp.sum(jnp.where(sel, VT, 0.0), axis=1, keepdims=True)
        tT = jnp.tanh(PUT - vcol).astype(bf16)
        return tT, bias

    def qloop(q, carry):
        acc, tT, bias = carry
        hT = jnp.dot(WbT, tT, preferred_element_type=f32)
        r = jnp.max(hT + bias, axis=1, keepdims=True) + bbT
        acc = jnp.where(iota_m == q, r, acc)
        tT2, bias2 = stage_a(q + 1)
        return acc, tT2, bias2

    acc0 = jnp.zeros((Co2, M), f32)
    tT0, bias0 = stage_a(0)
    acc_fin, _, _ = jax.lax.fori_loop(0, M, qloop, (acc0, tT0, bias0))
    out_ref[0] = acc_fin


def _sa(XT, posT, q, qT, Wa, ba, Wb, bb, r2):
    B, F, N = XT.shape
    M = qT.shape[2]
    Co2 = Wb.shape[1]
    waxT = Wa[:F].T
    wapT = Wa[F:].T
    baT = ba.reshape(-1, 1)
    wbT = Wb.T
    bbT = bb.reshape(-1, 1)
    return pl.pallas_call(
        functools.partial(_sa_body, r2=r2),
        grid=(B,),
        in_specs=[
            pl.BlockSpec((1, F, N), lambda b: (b, 0, 0)),
            pl.BlockSpec((1, 2, N), lambda b: (b, 0, 0)),
            pl.BlockSpec((1, M, 2), lambda b: (b, 0, 0)),
            pl.BlockSpec((1, 2, M), lambda b: (b, 0, 0)),
            pl.BlockSpec(waxT.shape, lambda b: (0, 0)),
            pl.BlockSpec(wapT.shape, lambda b: (0, 0)),
            pl.BlockSpec(baT.shape, lambda b: (0, 0)),
            pl.BlockSpec(wbT.shape, lambda b: (0, 0)),
            pl.BlockSpec(bbT.shape, lambda b: (0, 0)),
        ],
        out_specs=pl.BlockSpec((1, Co2, M), lambda b: (b, 0, 0)),
        out_shape=jax.ShapeDtypeStruct((B, Co2, M), jnp.float32),
        compiler_params=pltpu.CompilerParams(
            dimension_semantics=("parallel",)),
    )(XT, posT, q, qT, waxT, wapT, baT, wbT, bbT)


def _glob_body(xT_ref, qT_ref, waxT_ref, wapT_ref, baT_ref, wbT_ref, bbT_ref,
               out_ref):
    f32 = jnp.float32
    XT = xT_ref[0]
    QT = qT_ref[0]
    hT = jnp.tanh(jnp.dot(waxT_ref[...], XT, preferred_element_type=f32)
                  + jnp.dot(wapT_ref[...], QT, preferred_element_type=f32)
                  + baT_ref[...])
    oT = jnp.dot(wbT_ref[...], hT, preferred_element_type=f32) + bbT_ref[...]
    out_ref[0] = jnp.max(oT, axis=1, keepdims=True)


def _glob(XT, qT, Wa, ba, Wb, bb):
    B, C, M = XT.shape
    Co2 = Wb.shape[1]
    waxT = Wa[:C].T
    wapT = Wa[C:].T
    baT = ba.reshape(-1, 1)
    wbT = Wb.T
    bbT = bb.reshape(-1, 1)
    return pl.pallas_call(
        _glob_body,
        grid=(B,),
        in_specs=[
            pl.BlockSpec((1, C, M), lambda b: (b, 0, 0)),
            pl.BlockSpec((1, 2, M), lambda b: (b, 0, 0)),
            pl.BlockSpec(waxT.shape, lambda b: (0, 0)),
            pl.BlockSpec(wapT.shape, lambda b: (0, 0)),
            pl.BlockSpec(baT.shape, lambda b: (0, 0)),
            pl.BlockSpec(wbT.shape, lambda b: (0, 0)),
            pl.BlockSpec(bbT.shape, lambda b: (0, 0)),
        ],
        out_specs=pl.BlockSpec((1, Co2, 1), lambda b: (b, 0, 0)),
        out_shape=jax.ShapeDtypeStruct((B, Co2, 1), jnp.float32),
        compiler_params=pltpu.CompilerParams(
            dimension_semantics=("parallel",)),
    )(XT, qT, waxT, wapT, baT, wbT, bbT)[:, :, 0]


def kernel(x, pos, W1a, b1a, W1b, b1b, W2a, b2a, W2b, b2b, W3a, b3a, W3b, b3b):
    B, N, _ = x.shape
    M1 = N // 4
    M2 = M1 // 4
    r1sq = float(0.4 * 0.4)
    r2sq = float(0.8 * 0.8)

    px = pos[:, :, 0]
    py = pos[:, :, 1]
    xT = jnp.transpose(x, (0, 2, 1))
    posT = jnp.stack([px, py], axis=1)

    q1x, q1y = _fps(px, py, M1)
    q1 = jnp.stack([q1x, q1y], axis=-1)
    q1T = jnp.stack([q1x, q1y], axis=1)

    x1T = _sa(xT, posT, q1, q1T, W1a, b1a, W1b, b1b, r2=r1sq)

    q2x, q2y = _fps(q1x, q1y, M2)
    q2 = jnp.stack([q2x, q2y], axis=-1)
    q2T = jnp.stack([q2x, q2y], axis=1)

    x2T = _sa(x1T, q1T, q2, q2T, W2a, b2a, W2b, b2b, r2=r2sq)

    return _glob(x2T, q2T, W3a, b3a, W3b, b3b)


# speedup vs baseline: 1.6664x; 1.5385x over previous
import functools

import jax
import jax.numpy as jnp
from jax.experimental import pallas as pl
from jax.experimental.pallas import tpu as pltpu

_MAXK = 128
_BISECT_ITERS = 46


def _fps_body(px_ref, py_ref, qx_ref, qy_ref, *, M):
    px = px_ref[...]
    py = py_ref[...]
    B, N = px.shape
    iota_n = jax.lax.broadcasted_iota(jnp.int32, (1, N), 1)
    iota_m = jax.lax.broadcasted_iota(jnp.int32, (1, M), 1)
    lastx = px[:, 0:1]
    lasty = py[:, 0:1]
    sel0 = iota_m == 0
    qx = jnp.where(sel0, lastx, 0.0)
    qy = jnp.where(sel0, lasty, 0.0)
    dist0 = jnp.full((B, N), jnp.inf, dtype=jnp.float32)

    def body(i, carry):
        dist, lx, ly, qx, qy = carry
        d = (px - lx) ** 2 + (py - ly) ** 2
        dist = jnp.minimum(dist, d)
        m = jnp.max(dist, axis=1, keepdims=True)
        idx = jnp.min(jnp.where(dist == m, iota_n, N), axis=1, keepdims=True)
        selp = iota_n == idx
        nx = jnp.sum(jnp.where(selp, px, 0.0), axis=1, keepdims=True)
        ny = jnp.sum(jnp.where(selp, py, 0.0), axis=1, keepdims=True)
        selq = iota_m == i
        qx = jnp.where(selq, nx, qx)
        qy = jnp.where(selq, ny, qy)
        return dist, nx, ny, qx, qy

    _, _, _, qx, qy = jax.lax.fori_loop(1, M, body, (dist0, lastx, lasty, qx, qy))
    qx_ref[...] = qx
    qy_ref[...] = qy


def _fps(px, py, M):
    B, N = px.shape
    return pl.pallas_call(
        functools.partial(_fps_body, M=M),
        out_shape=[
            jax.ShapeDtypeStruct((B, M), jnp.float32),
            jax.ShapeDtypeStruct((B, M), jnp.float32),
        ],
    )(px, py)


def _sa_body(xT_ref, posT_ref, q_ref, qT_ref, waxT_ref, wapT_ref, baT_ref,
             wbT_ref, bbT_ref, out_ref, *, r2):
    XT = xT_ref[0]
    PT = posT_ref[0]
    Q = q_ref[0]
    QT = qT_ref[0]
    N = XT.shape[1]
    M = QT.shape[1]
    f32 = jnp.float32

    PUT = (jnp.dot(waxT_ref[...], XT, preferred_element_type=f32)
           + jnp.dot(wapT_ref[...], PT, preferred_element_type=f32)
           + baT_ref[...])
    VT = jnp.dot(wapT_ref[...], QT, preferred_element_type=f32)

    PTx = PT[0:1, :]
    PTy = PT[1:2, :]
    QTx = QT[0:1, :]
    QTy = QT[1:2, :]
    Qx = Q[:, 0:1]
    Qy = Q[:, 1:2]

    ddx = Qx - PTx
    ddy = Qy - PTy
    D = ddx * ddx + ddy * ddy
    cnt = jnp.sum((D <= r2).astype(jnp.int32), axis=1, keepdims=True)

    def bis(_, c):
        lo, hi = c
        mid = 0.5 * (lo + hi)
        cm = jnp.sum((D <= mid).astype(jnp.int32), axis=1, keepdims=True)
        ge = cm >= _MAXK
        return jnp.where(ge, lo, mid), jnp.where(ge, mid, hi)

    lo0 = jnp.zeros((M, 1), f32)
    hi0 = jnp.full((M, 1), r2, f32)
    _, hi = jax.lax.fori_loop(0, _BISECT_ITERS, bis, (lo0, hi0))
    thresh = jnp.where(cnt > _MAXK, hi, jnp.full((M, 1), r2, f32))

    WbT = wbT_ref[...].astype(jnp.bfloat16)
    bbT = bbT_ref[...]
    Co2 = WbT.shape[0]
    bf16 = jnp.bfloat16
    iota_m = jax.lax.broadcasted_iota(jnp.int32, (1, M), 1)
    iota_mc = jax.lax.broadcasted_iota(jnp.int32, (M, 1), 0)

    QC = 16

    def chunk(c, acc):
        base = c * QC
        pieces_t = []
        pieces_b = []
        for k in range(QC):
            q = base + k
            sel = iota_m == q
            qx = jnp.sum(jnp.where(sel, QTx, 0.0))
            qy = jnp.sum(jnp.where(sel, QTy, 0.0))
            th = jnp.sum(jnp.where(iota_mc == q, thresh, 0.0))
            ex = qx - PTx
            ey = qy - PTy
            pieces_b.append(jnp.where(ex * ex + ey * ey <= th, 0.0, -1e30))
            vcol = jnp.sum(jnp.where(sel, VT, 0.0), axis=1, keepdims=True)
            pieces_t.append(jnp.tanh(PUT - vcol).astype(bf16))
        tS = jnp.concatenate(pieces_t, axis=1)
        bS = jnp.concatenate(pieces_b, axis=1)
        hS = jnp.dot(WbT, tS, preferred_element_type=f32) + bS
        for k in range(QC):
            r = jnp.max(hS[:, k * N:(k + 1) * N], axis=1, keepdims=True) + bbT
            acc = jnp.where(iota_m == (base + k), r, acc)
        return acc

    acc0 = jnp.zeros((Co2, M), f32)
    out_ref[0] = jax.lax.fori_loop(0, M // QC, chunk, acc0)


def _sa(XT, posT, q, qT, Wa, ba, Wb, bb, r2):
    B, F, N = XT.shape
    M = qT.shape[2]
    Co2 = Wb.shape[1]
    waxT = Wa[:F].T
    wapT = Wa[F:].T
    baT = ba.reshape(-1, 1)
    wbT = Wb.T
    bbT = bb.reshape(-1, 1)
    return pl.pallas_call(
        functools.partial(_sa_body, r2=r2),
        grid=(B,),
        in_specs=[
            pl.BlockSpec((1, F, N), lambda b: (b, 0, 0)),
            pl.BlockSpec((1, 2, N), lambda b: (b, 0, 0)),
            pl.BlockSpec((1, M, 2), lambda b: (b, 0, 0)),
            pl.BlockSpec((1, 2, M), lambda b: (b, 0, 0)),
            pl.BlockSpec(waxT.shape, lambda b: (0, 0)),
            pl.BlockSpec(wapT.shape, lambda b: (0, 0)),
            pl.BlockSpec(baT.shape, lambda b: (0, 0)),
            pl.BlockSpec(wbT.shape, lambda b: (0, 0)),
            pl.BlockSpec(bbT.shape, lambda b: (0, 0)),
        ],
        out_specs=pl.BlockSpec((1, Co2, M), lambda b: (b, 0, 0)),
        out_shape=jax.ShapeDtypeStruct((B, Co2, M), jnp.float32),
        compiler_params=pltpu.CompilerParams(
            dimension_semantics=("parallel",)),
    )(XT, posT, q, qT, waxT, wapT, baT, wbT, bbT)


def _glob_body(xT_ref, qT_ref, waxT_ref, wapT_ref, baT_ref, wbT_ref, bbT_ref,
               out_ref):
    f32 = jnp.float32
    XT = xT_ref[0]
    QT = qT_ref[0]
    hT = jnp.tanh(jnp.dot(waxT_ref[...], XT, preferred_element_type=f32)
                  + jnp.dot(wapT_ref[...], QT, preferred_element_type=f32)
                  + baT_ref[...])
    oT = jnp.dot(wbT_ref[...], hT, preferred_element_type=f32) + bbT_ref[...]
    out_ref[0] = jnp.max(oT, axis=1, keepdims=True)


def _glob(XT, qT, Wa, ba, Wb, bb):
    B, C, M = XT.shape
    Co2 = Wb.shape[1]
    waxT = Wa[:C].T
    wapT = Wa[C:].T
    baT = ba.reshape(-1, 1)
    wbT = Wb.T
    bbT = bb.reshape(-1, 1)
    return pl.pallas_call(
        _glob_body,
        grid=(B,),
        in_specs=[
            pl.BlockSpec((1, C, M), lambda b: (b, 0, 0)),
            pl.BlockSpec((1, 2, M), lambda b: (b, 0, 0)),
            pl.BlockSpec(waxT.shape, lambda b: (0, 0)),
            pl.BlockSpec(wapT.shape, lambda b: (0, 0)),
            pl.BlockSpec(baT.shape, lambda b: (0, 0)),
            pl.BlockSpec(wbT.shape, lambda b: (0, 0)),
            pl.BlockSpec(bbT.shape, lambda b: (0, 0)),
        ],
        out_specs=pl.BlockSpec((1, Co2, 1), lambda b: (b, 0, 0)),
        out_shape=jax.ShapeDtypeStruct((B, Co2, 1), jnp.float32),
        compiler_params=pltpu.CompilerParams(
            dimension_semantics=("parallel",)),
    )(XT, qT, waxT, wapT, baT, wbT, bbT)[:, :, 0]


def kernel(x, pos, W1a, b1a, W1b, b1b, W2a, b2a, W2b, b2b, W3a, b3a, W3b, b3b):
    B, N, _ = x.shape
    M1 = N // 4
    M2 = M1 // 4
    r1sq = float(0.4 * 0.4)
    r2sq = float(0.8 * 0.8)

    px = pos[:, :, 0]
    py = pos[:, :, 1]
    xT = jnp.transpose(x, (0, 2, 1))
    posT = jnp.stack([px, py], axis=1)

    q1x, q1y = _fps(px, py, M1)
    q1 = jnp.stack([q1x, q1y], axis=-1)
    q1T = jnp.stack([q1x, q1y], axis=1)

    x1T = _sa(xT, posT, q1, q1T, W1a, b1a, W1b, b1b, r2=r1sq)

    q2x, q2y = _fps(q1x, q1y, M2)
    q2 = jnp.stack([q2x, q2y], axis=-1)
    q2T = jnp.stack([q2x, q2y], axis=1)

    x2T = _sa(x1T, q1T, q2, q2T, W2a, b2a, W2b, b2b, r2=r2sq)

    return _glob(x2T, q2T, W3a, b3a, W3b, b3b)


# speedup vs baseline: 1.9318x; 1.1593x over previous
import functools

import jax
import jax.numpy as jnp
from jax.experimental import pallas as pl
from jax.experimental.pallas import tpu as pltpu

_MAXK = 128
_BISECT_ITERS = 46


def _fps_body(px_ref, py_ref, qx_ref, qy_ref, *, M):
    px = px_ref[...]
    py = py_ref[...]
    B, N = px.shape
    iota_n = jax.lax.broadcasted_iota(jnp.int32, (1, N), 1)
    iota_m = jax.lax.broadcasted_iota(jnp.int32, (1, M), 1)
    lastx = px[:, 0:1]
    lasty = py[:, 0:1]
    sel0 = iota_m == 0
    qx = jnp.where(sel0, lastx, 0.0)
    qy = jnp.where(sel0, lasty, 0.0)
    dist0 = jnp.full((B, N), jnp.inf, dtype=jnp.float32)

    def body(i, carry):
        dist, lx, ly, qx, qy = carry
        d = (px - lx) ** 2 + (py - ly) ** 2
        dist = jnp.minimum(dist, d)
        m = jnp.max(dist, axis=1, keepdims=True)
        idx = jnp.min(jnp.where(dist == m, iota_n, N), axis=1, keepdims=True)
        selp = iota_n == idx
        nx = jnp.sum(jnp.where(selp, px, 0.0), axis=1, keepdims=True)
        ny = jnp.sum(jnp.where(selp, py, 0.0), axis=1, keepdims=True)
        selq = iota_m == i
        qx = jnp.where(selq, nx, qx)
        qy = jnp.where(selq, ny, qy)
        return dist, nx, ny, qx, qy

    _, _, _, qx, qy = jax.lax.fori_loop(1, M, body, (dist0, lastx, lasty, qx, qy))
    qx_ref[...] = qx
    qy_ref[...] = qy


def _fps(px, py, M):
    B, N = px.shape
    return pl.pallas_call(
        functools.partial(_fps_body, M=M),
        out_shape=[
            jax.ShapeDtypeStruct((B, M), jnp.float32),
            jax.ShapeDtypeStruct((B, M), jnp.float32),
        ],
    )(px, py)


def _sa_body(xT_ref, posT_ref, q_ref, qT_ref, waxT_ref, wapT_ref, baT_ref,
             wbT_ref, bbT_ref, out_ref, *, r2):
    XT = xT_ref[0]
    PT = posT_ref[0]
    Q = q_ref[0]
    QT = qT_ref[0]
    N = XT.shape[1]
    M = QT.shape[1]
    f32 = jnp.float32

    PUT = (jnp.dot(waxT_ref[...], XT, preferred_element_type=f32)
           + jnp.dot(wapT_ref[...], PT, preferred_element_type=f32)
           + baT_ref[...])
    VT = jnp.dot(wapT_ref[...], QT, preferred_element_type=f32)

    PTx = PT[0:1, :]
    PTy = PT[1:2, :]
    QTx = QT[0:1, :]
    QTy = QT[1:2, :]
    Qx = Q[:, 0:1]
    Qy = Q[:, 1:2]

    ddx = Qx - PTx
    ddy = Qy - PTy
    D = ddx * ddx + ddy * ddy
    cnt = jnp.sum((D <= r2).astype(jnp.int32), axis=1, keepdims=True)

    def bis(_, c):
        lo, hi = c
        mid = 0.5 * (lo + hi)
        cm = jnp.sum((D <= mid).astype(jnp.int32), axis=1, keepdims=True)
        ge = cm >= _MAXK
        return jnp.where(ge, lo, mid), jnp.where(ge, mid, hi)

    lo0 = jnp.zeros((M, 1), f32)
    hi0 = jnp.full((M, 1), r2, f32)
    _, hi = jax.lax.fori_loop(0, _BISECT_ITERS, bis, (lo0, hi0))
    thresh = jnp.where(cnt > _MAXK, hi, jnp.full((M, 1), r2, f32))

    WbT = wbT_ref[...].astype(jnp.bfloat16)
    bbT = bbT_ref[...]
    Co2 = WbT.shape[0]
    bf16 = jnp.bfloat16
    iota_m = jax.lax.broadcasted_iota(jnp.int32, (1, M), 1)
    iota_mc = jax.lax.broadcasted_iota(jnp.int32, (M, 1), 0)

    QC = 32

    def chunk(c, acc):
        base = c * QC
        pieces_t = []
        pieces_b = []
        for k in range(QC):
            q = base + k
            sel = iota_m == q
            qx = jnp.sum(jnp.where(sel, QTx, 0.0))
            qy = jnp.sum(jnp.where(sel, QTy, 0.0))
            th = jnp.sum(jnp.where(iota_mc == q, thresh, 0.0))
            ex = qx - PTx
            ey = qy - PTy
            pieces_b.append(jnp.where(ex * ex + ey * ey <= th, 0.0, -1e30))
            vcol = jnp.sum(jnp.where(sel, VT, 0.0), axis=1, keepdims=True)
            pieces_t.append(jnp.tanh(PUT - vcol).astype(bf16))
        tS = jnp.concatenate(pieces_t, axis=1)
        bS = jnp.concatenate(pieces_b, axis=1)
        hS = jnp.dot(WbT, tS, preferred_element_type=f32) + bS
        for k in range(QC):
            r = jnp.max(hS[:, k * N:(k + 1) * N], axis=1, keepdims=True) + bbT
            acc = jnp.where(iota_m == (base + k), r, acc)
        return acc

    acc0 = jnp.zeros((Co2, M), f32)
    out_ref[0] = jax.lax.fori_loop(0, M // QC, chunk, acc0)


def _sa(XT, posT, q, qT, Wa, ba, Wb, bb, r2):
    B, F, N = XT.shape
    M = qT.shape[2]
    Co2 = Wb.shape[1]
    waxT = Wa[:F].T
    wapT = Wa[F:].T
    baT = ba.reshape(-1, 1)
    wbT = Wb.T
    bbT = bb.reshape(-1, 1)
    return pl.pallas_call(
        functools.partial(_sa_body, r2=r2),
        grid=(B,),
        in_specs=[
            pl.BlockSpec((1, F, N), lambda b: (b, 0, 0)),
            pl.BlockSpec((1, 2, N), lambda b: (b, 0, 0)),
            pl.BlockSpec((1, M, 2), lambda b: (b, 0, 0)),
            pl.BlockSpec((1, 2, M), lambda b: (b, 0, 0)),
            pl.BlockSpec(waxT.shape, lambda b: (0, 0)),
            pl.BlockSpec(wapT.shape, lambda b: (0, 0)),
            pl.BlockSpec(baT.shape, lambda b: (0, 0)),
            pl.BlockSpec(wbT.shape, lambda b: (0, 0)),
            pl.BlockSpec(bbT.shape, lambda b: (0, 0)),
        ],
        out_specs=pl.BlockSpec((1, Co2, M), lambda b: (b, 0, 0)),
        out_shape=jax.ShapeDtypeStruct((B, Co2, M), jnp.float32),
        compiler_params=pltpu.CompilerParams(
            dimension_semantics=("parallel",)),
    )(XT, posT, q, qT, waxT, wapT, baT, wbT, bbT)


def _glob_body(xT_ref, qT_ref, waxT_ref, wapT_ref, baT_ref, wbT_ref, bbT_ref,
               out_ref):
    f32 = jnp.float32
    XT = xT_ref[0]
    QT = qT_ref[0]
    hT = jnp.tanh(jnp.dot(waxT_ref[...], XT, preferred_element_type=f32)
                  + jnp.dot(wapT_ref[...], QT, preferred_element_type=f32)
                  + baT_ref[...])
    oT = jnp.dot(wbT_ref[...], hT, preferred_element_type=f32) + bbT_ref[...]
    out_ref[0] = jnp.max(oT, axis=1, keepdims=True)


def _glob(XT, qT, Wa, ba, Wb, bb):
    B, C, M = XT.shape
    Co2 = Wb.shape[1]
    waxT = Wa[:C].T
    wapT = Wa[C:].T
    baT = ba.reshape(-1, 1)
    wbT = Wb.T
    bbT = bb.reshape(-1, 1)
    return pl.pallas_call(
        _glob_body,
        grid=(B,),
        in_specs=[
            pl.BlockSpec((1, C, M), lambda b: (b, 0, 0)),
            pl.BlockSpec((1, 2, M), lambda b: (b, 0, 0)),
            pl.BlockSpec(waxT.shape, lambda b: (0, 0)),
            pl.BlockSpec(wapT.shape, lambda b: (0, 0)),
            pl.BlockSpec(baT.shape, lambda b: (0, 0)),
            pl.BlockSpec(wbT.shape, lambda b: (0, 0)),
            pl.BlockSpec(bbT.shape, lambda b: (0, 0)),
        ],
        out_specs=pl.BlockSpec((1, Co2, 1), lambda b: (b, 0, 0)),
        out_shape=jax.ShapeDtypeStruct((B, Co2, 1), jnp.float32),
        compiler_params=pltpu.CompilerParams(
            dimension_semantics=("parallel",)),
    )(XT, qT, waxT, wapT, baT, wbT, bbT)[:, :, 0]


def kernel(x, pos, W1a, b1a, W1b, b1b, W2a, b2a, W2b, b2b, W3a, b3a, W3b, b3b):
    B, N, _ = x.shape
    M1 = N // 4
    M2 = M1 // 4
    r1sq = float(0.4 * 0.4)
    r2sq = float(0.8 * 0.8)

    px = pos[:, :, 0]
    py = pos[:, :, 1]
    xT = jnp.transpose(x, (0, 2, 1))
    posT = jnp.stack([px, py], axis=1)

    q1x, q1y = _fps(px, py, M1)
    q1 = jnp.stack([q1x, q1y], axis=-1)
    q1T = jnp.stack([q1x, q1y], axis=1)

    x1T = _sa(xT, posT, q1, q1T, W1a, b1a, W1b, b1b, r2=r1sq)

    q2x, q2y = _fps(q1x, q1y, M2)
    q2 = jnp.stack([q2x, q2y], axis=-1)
    q2T = jnp.stack([q2x, q2y], axis=1)

    x2T = _sa(x1T, q1T, q2, q2T, W2a, b2a, W2b, b2b, r2=r2sq)

    return _glob(x2T, q2T, W3a, b3a, W3b, b3b)


# speedup vs baseline: 1.9404x; 1.0044x over previous
import functools

import jax
import jax.numpy as jnp
from jax.experimental import pallas as pl
from jax.experimental.pallas import tpu as pltpu

_MAXK = 128
_BISECT_ITERS = 46


def _fps_body(px_ref, py_ref, qx_ref, qy_ref, *, M):
    px = px_ref[...]
    py = py_ref[...]
    B, N = px.shape
    iota_n = jax.lax.broadcasted_iota(jnp.int32, (1, N), 1)
    iota_m = jax.lax.broadcasted_iota(jnp.int32, (1, M), 1)
    lastx = px[:, 0:1]
    lasty = py[:, 0:1]
    sel0 = iota_m == 0
    qx = jnp.where(sel0, lastx, 0.0)
    qy = jnp.where(sel0, lasty, 0.0)
    dist0 = jnp.full((B, N), jnp.inf, dtype=jnp.float32)

    def body(i, carry):
        dist, lx, ly, qx, qy = carry
        d = (px - lx) ** 2 + (py - ly) ** 2
        dist = jnp.minimum(dist, d)
        m = jnp.max(dist, axis=1, keepdims=True)
        idx = jnp.min(jnp.where(dist == m, iota_n, N), axis=1, keepdims=True)
        selp = iota_n == idx
        nx = jnp.sum(jnp.where(selp, px, 0.0), axis=1, keepdims=True)
        ny = jnp.sum(jnp.where(selp, py, 0.0), axis=1, keepdims=True)
        selq = iota_m == i
        qx = jnp.where(selq, nx, qx)
        qy = jnp.where(selq, ny, qy)
        return dist, nx, ny, qx, qy

    _, _, _, qx, qy = jax.lax.fori_loop(1, M, body, (dist0, lastx, lasty, qx, qy))
    qx_ref[...] = qx
    qy_ref[...] = qy


def _fps(px, py, M):
    B, N = px.shape
    return pl.pallas_call(
        functools.partial(_fps_body, M=M),
        out_shape=[
            jax.ShapeDtypeStruct((B, M), jnp.float32),
            jax.ShapeDtypeStruct((B, M), jnp.float32),
        ],
    )(px, py)


def _sa_body(xT_ref, posT_ref, q_ref, qT_ref, waxT_ref, wapT_ref, baT_ref,
             wbT_ref, bbT_ref, out_ref, *, r2):
    XT = xT_ref[0]
    PT = posT_ref[0]
    Q = q_ref[0]
    QT = qT_ref[0]
    N = XT.shape[1]
    M = QT.shape[1]
    f32 = jnp.float32

    PUT = (jnp.dot(waxT_ref[...], XT, preferred_element_type=f32)
           + jnp.dot(wapT_ref[...], PT, preferred_element_type=f32)
           + baT_ref[...])
    VT = jnp.dot(wapT_ref[...], QT, preferred_element_type=f32)

    PTx = PT[0:1, :]
    PTy = PT[1:2, :]
    QTx = QT[0:1, :]
    QTy = QT[1:2, :]
    Qx = Q[:, 0:1]
    Qy = Q[:, 1:2]

    ddx = Qx - PTx
    ddy = Qy - PTy
    D = ddx * ddx + ddy * ddy
    cnt = jnp.sum((D <= r2).astype(jnp.int32), axis=1, keepdims=True)

    def bis(_, c):
        lo, hi = c
        mid = 0.5 * (lo + hi)
        cm = jnp.sum((D <= mid).astype(jnp.int32), axis=1, keepdims=True)
        ge = cm >= _MAXK
        return jnp.where(ge, lo, mid), jnp.where(ge, mid, hi)

    lo0 = jnp.zeros((M, 1), f32)
    hi0 = jnp.full((M, 1), r2, f32)
    _, hi = jax.lax.fori_loop(0, _BISECT_ITERS, bis, (lo0, hi0))
    thresh = jnp.where(cnt > _MAXK, hi, jnp.full((M, 1), r2, f32))

    WbT = wbT_ref[...].astype(jnp.bfloat16)
    bbT = bbT_ref[...]
    Co2 = WbT.shape[0]
    bf16 = jnp.bfloat16
    iota_m = jax.lax.broadcasted_iota(jnp.int32, (1, M), 1)
    iota_mc = jax.lax.broadcasted_iota(jnp.int32, (M, 1), 0)

    QC = 64

    def chunk(c, acc):
        base = c * QC
        pieces_t = []
        pieces_b = []
        for k in range(QC):
            q = base + k
            sel = iota_m == q
            qx = jnp.sum(jnp.where(sel, QTx, 0.0))
            qy = jnp.sum(jnp.where(sel, QTy, 0.0))
            th = jnp.sum(jnp.where(iota_mc == q, thresh, 0.0))
            ex = qx - PTx
            ey = qy - PTy
            pieces_b.append(jnp.where(ex * ex + ey * ey <= th, 0.0, -1e30))
            vcol = jnp.sum(jnp.where(sel, VT, 0.0), axis=1, keepdims=True)
            pieces_t.append(jnp.tanh(PUT - vcol).astype(bf16))
        tS = jnp.concatenate(pieces_t, axis=1)
        bS = jnp.concatenate(pieces_b, axis=1)
        hS = jnp.dot(WbT, tS, preferred_element_type=f32) + bS
        for k in range(QC):
            r = jnp.max(hS[:, k * N:(k + 1) * N], axis=1, keepdims=True) + bbT
            acc = jnp.where(iota_m == (base + k), r, acc)
        return acc

    acc0 = jnp.zeros((Co2, M), f32)
    out_ref[0] = jax.lax.fori_loop(0, M // QC, chunk, acc0)


def _sa(XT, posT, q, qT, Wa, ba, Wb, bb, r2):
    B, F, N = XT.shape
    M = qT.shape[2]
    Co2 = Wb.shape[1]
    waxT = Wa[:F].T
    wapT = Wa[F:].T
    baT = ba.reshape(-1, 1)
    wbT = Wb.T
    bbT = bb.reshape(-1, 1)
    return pl.pallas_call(
        functools.partial(_sa_body, r2=r2),
        grid=(B,),
        in_specs=[
            pl.BlockSpec((1, F, N), lambda b: (b, 0, 0)),
            pl.BlockSpec((1, 2, N), lambda b: (b, 0, 0)),
            pl.BlockSpec((1, M, 2), lambda b: (b, 0, 0)),
            pl.BlockSpec((1, 2, M), lambda b: (b, 0, 0)),
            pl.BlockSpec(waxT.shape, lambda b: (0, 0)),
            pl.BlockSpec(wapT.shape, lambda b: (0, 0)),
            pl.BlockSpec(baT.shape, lambda b: (0, 0)),
            pl.BlockSpec(wbT.shape, lambda b: (0, 0)),
            pl.BlockSpec(bbT.shape, lambda b: (0, 0)),
        ],
        out_specs=pl.BlockSpec((1, Co2, M), lambda b: (b, 0, 0)),
        out_shape=jax.ShapeDtypeStruct((B, Co2, M), jnp.float32),
        compiler_params=pltpu.CompilerParams(
            dimension_semantics=("parallel",)),
    )(XT, posT, q, qT, waxT, wapT, baT, wbT, bbT)


def _glob_body(xT_ref, qT_ref, waxT_ref, wapT_ref, baT_ref, wbT_ref, bbT_ref,
               out_ref):
    f32 = jnp.float32
    XT = xT_ref[0]
    QT = qT_ref[0]
    hT = jnp.tanh(jnp.dot(waxT_ref[...], XT, preferred_element_type=f32)
                  + jnp.dot(wapT_ref[...], QT, preferred_element_type=f32)
                  + baT_ref[...])
    oT = jnp.dot(wbT_ref[...], hT, preferred_element_type=f32) + bbT_ref[...]
    out_ref[0] = jnp.max(oT, axis=1, keepdims=True)


def _glob(XT, qT, Wa, ba, Wb, bb):
    B, C, M = XT.shape
    Co2 = Wb.shape[1]
    waxT = Wa[:C].T
    wapT = Wa[C:].T
    baT = ba.reshape(-1, 1)
    wbT = Wb.T
    bbT = bb.reshape(-1, 1)
    return pl.pallas_call(
        _glob_body,
        grid=(B,),
        in_specs=[
            pl.BlockSpec((1, C, M), lambda b: (b, 0, 0)),
            pl.BlockSpec((1, 2, M), lambda b: (b, 0, 0)),
            pl.BlockSpec(waxT.shape, lambda b: (0, 0)),
            pl.BlockSpec(wapT.shape, lambda b: (0, 0)),
            pl.BlockSpec(baT.shape, lambda b: (0, 0)),
            pl.BlockSpec(wbT.shape, lambda b: (0, 0)),
            pl.BlockSpec(bbT.shape, lambda b: (0, 0)),
        ],
        out_specs=pl.BlockSpec((1, Co2, 1), lambda b: (b, 0, 0)),
        out_shape=jax.ShapeDtypeStruct((B, Co2, 1), jnp.float32),
        compiler_params=pltpu.CompilerParams(
            dimension_semantics=("parallel",)),
    )(XT, qT, waxT, wapT, baT, wbT, bbT)[:, :, 0]


def kernel(x, pos, W1a, b1a, W1b, b1b, W2a, b2a, W2b, b2b, W3a, b3a, W3b, b3b):
    B, N, _ = x.shape
    M1 = N // 4
    M2 = M1 // 4
    r1sq = float(0.4 * 0.4)
    r2sq = float(0.8 * 0.8)

    px = pos[:, :, 0]
    py = pos[:, :, 1]
    xT = jnp.transpose(x, (0, 2, 1))
    posT = jnp.stack([px, py], axis=1)

    q1x, q1y = _fps(px, py, M1)
    q1 = jnp.stack([q1x, q1y], axis=-1)
    q1T = jnp.stack([q1x, q1y], axis=1)

    x1T = _sa(xT, posT, q1, q1T, W1a, b1a, W1b, b1b, r2=r1sq)

    q2x, q2y = _fps(q1x, q1y, M2)
    q2 = jnp.stack([q2x, q2y], axis=-1)
    q2T = jnp.stack([q2x, q2y], axis=1)

    x2T = _sa(x1T, q1T, q2, q2T, W2a, b2a, W2b, b2b, r2=r2sq)

    return _glob(x2T, q2T, W3a, b3a, W3b, b3b)
